# baseline (speedup 1.0000x reference)
import jax
import jax.numpy as jnp
from jax import lax
from jax.experimental import pallas as pl
from jax.experimental.pallas import tpu as pltpu
from jax.experimental.pallas import tpu_sc as plsc

B, S, D = 1024, 512, 128
N = B * S
NC, NS = 2, 16
NW = NC * NS
TOK_PER_TILE = N // NW
T = 64
CHUNKS = TOK_PER_TILE // T
DV = D // 16
TU = 2


def _rsqrt(x):
    i = lax.bitcast_convert_type(x, jnp.int32)
    i = 0x5F3759DF - lax.shift_right_arithmetic(i, 1)
    y = lax.bitcast_convert_type(i, jnp.float32)
    for _ in range(3):
        y = y * (1.5 - 0.5 * x * y * y)
    return y


def _tree_sum(vs):
    while len(vs) > 1:
        vs = [vs[i] + vs[i + 1] for i in range(0, len(vs), 2)]
    return vs[0]


def _body(ids_hbm, tt_hbm, word_hbm, pos_hbm, ttrow_hbm, gamma_hbm, beta_hbm,
          out_hbm,
          idx0, idx1, ttv0, ttv1, rows0, rows1, ob0, ob1,
          pos_v, ttrow_v, gamma_v, beta_v,
          gsem0, gsem1, isem0, isem1, osem0, osem1):
    idx = (idx0, idx1)
    ttv = (ttv0, ttv1)
    rows = (rows0, rows1)
    obuf = (ob0, ob1)
    gsem = (gsem0, gsem1)
    isem = (isem0, isem1)
    osem = (osem0, osem1)

    wid = lax.axis_index("s") * NC + lax.axis_index("c")
    base_tile = wid * TOK_PER_TILE

    pltpu.sync_copy(pos_hbm, pos_v)
    pltpu.sync_copy(ttrow_hbm, ttrow_v)
    pltpu.sync_copy(gamma_hbm, gamma_v)
    pltpu.sync_copy(beta_hbm, beta_v)

    def ids_copies(c, b):
        base = base_tile + c * T
        return (pltpu.make_async_copy(ids_hbm.at[pl.ds(base, T)],
                                      idx[b], isem[b]),
                pltpu.make_async_copy(tt_hbm.at[pl.ds(base, T)],
                                      ttv[b].at[pl.ds(0, T)], isem[b]))

    def gather(b):
        return pltpu.make_async_copy(word_hbm.at[idx[b]], rows[b], gsem[b])

    def outcp(c, b):
        base = base_tile + c * T
        return pltpu.make_async_copy(obuf[b], out_hbm.at[pl.ds(base, T)],
                                     osem[b])

    iota16 = lax.iota(jnp.int32, 16)
    perms = [iota16 ^ off for off in (8, 4, 2, 1)]

    permdn = lax.GatherDimensionNumbers(
        offset_dims=(), collapsed_slice_dims=(0,), start_index_map=(0,))

    def _lane_sum(v):
        for p in perms:
            v = v + lax.gather(v, p[:, None], permdn, slice_sizes=(1,),
                               mode=lax.GatherScatterMode.PROMISE_IN_BOUNDS)
        return v

    def compute(b, c):
        rows_v, ttv_v, ob_v = rows[b], ttv[b], obuf[b]
        s_off = lax.rem(c * T, S)
        gammas = [gamma_v[pl.ds(16 * k, 16)] for k in range(DV)]
        betas = [beta_v[pl.ds(16 * k, 16)] for k in range(DV)]

        @plsc.parallel_loop(0, T, step=TU)
        def _(i0):
            tgrp = ttv_v[pl.ds(i0, 16)]
            for u in range(TU):
                i = i0 + u
                t = tgrp[u]
                s_row = s_off + i
                xs = []
                for k in range(DV):
                    sl = pl.ds(16 * k, 16)
                    x = rows_v[i, sl] + pos_v[s_row, sl] + ttrow_v[t, sl]
                    xs.append(x)
                s1 = _lane_sum(_tree_sum(xs))
                s2 = _lane_sum(_tree_sum([x * x for x in xs]))
                mean = s1 * (1.0 / D)
                var = s2 * (1.0 / D) - mean * mean
                a = _rsqrt(var + 1e-5)
                sh = -mean * a
                for k in range(DV):
                    y = (xs[k] * a + sh) * gammas[k] + betas[k]
                    ob_v[i, pl.ds(16 * k, 16)] = y

    def step(c, b):
        nb = 1 - b

        @pl.when(c + 1 < CHUNKS)
        def _():
            for cp in ids_copies(c + 1, nb):
                cp.wait()
            gather(nb).start()

        gather(b).wait()

        @pl.when(c >= 2)
        def _():
            outcp(c - 2, b).wait()

        compute(b, c)

        @pl.when(c + 2 < CHUNKS)
        def _():
            for cp in ids_copies(c + 2, b):
                cp.start()

        outcp(c, b).start()

    for b in (0, 1):
        for cp in ids_copies(b, b):
            cp.start()
    for cp in ids_copies(0, 0):
        cp.wait()
    gather(0).start()

    def loop_body(i, carry):
        step(2 * i, 0)
        step(2 * i + 1, 1)
        return carry

    lax.fori_loop(0, CHUNKS // 2, loop_body, 0)
    outcp(CHUNKS - 2, 0).wait()
    outcp(CHUNKS - 1, 1).wait()


def kernel(input_ids, token_type_ids, word_table, pos_table, tt_table,
           ln_gamma, ln_beta):
    ids = input_ids.reshape(N).astype(jnp.int32)
    tts = token_type_ids.reshape(N).astype(jnp.int32)

    mesh = plsc.VectorSubcoreMesh(core_axis_name="c", subcore_axis_name="s",
                                  num_cores=NC, num_subcores=NS)
    fn = pl.kernel(
        _body,
        out_type=jax.ShapeDtypeStruct((N, D), jnp.float32),
        mesh=mesh,
        scratch_types=[
            pltpu.VMEM((T,), jnp.int32),
            pltpu.VMEM((T,), jnp.int32),
            pltpu.VMEM((T + 16,), jnp.int32),
            pltpu.VMEM((T + 16,), jnp.int32),
            pltpu.VMEM((T, D), jnp.float32),
            pltpu.VMEM((T, D), jnp.float32),
            pltpu.VMEM((T, D), jnp.float32),
            pltpu.VMEM((T, D), jnp.float32),
            pltpu.VMEM((S, D), jnp.float32),
            pltpu.VMEM((2, D), jnp.float32),
            pltpu.VMEM((D,), jnp.float32),
            pltpu.VMEM((D,), jnp.float32),
            pltpu.SemaphoreType.DMA,
            pltpu.SemaphoreType.DMA,
            pltpu.SemaphoreType.DMA,
            pltpu.SemaphoreType.DMA,
            pltpu.SemaphoreType.DMA,
            pltpu.SemaphoreType.DMA,
        ],
        compiler_params=pltpu.CompilerParams(needs_layout_passes=False),
    )
    out = fn(ids, tts, word_table, pos_table, tt_table, ln_gamma, ln_beta)
    return out.reshape(B, S, D)


# speedup vs baseline: 1.2344x; 1.2344x over previous
import jax
import jax.numpy as jnp
from jax import lax
from jax.experimental import pallas as pl
from jax.experimental.pallas import tpu as pltpu
from jax.experimental.pallas import tpu_sc as plsc

B, S, D = 1024, 512, 128
N = B * S
NC, NS = 2, 16
NW = NC * NS
TOK_PER_TILE = N // NW
T = 64
CHUNKS = TOK_PER_TILE // T
DV = D // 16
TU = 2


def _rsqrt(x):
    i = lax.bitcast_convert_type(x, jnp.int32)
    i = 0x5F3759DF - lax.shift_right_arithmetic(i, 1)
    y = lax.bitcast_convert_type(i, jnp.float32)
    for _ in range(3):
        y = y * (1.5 - 0.5 * x * y * y)
    return y


def _tree_sum(vs):
    while len(vs) > 1:
        vs = [vs[i] + vs[i + 1] for i in range(0, len(vs), 2)]
    return vs[0]


def _body(ids_hbm, tt_hbm, word_hbm, pos_hbm, ttrow_hbm, gamma_hbm, beta_hbm,
          out_hbm,
          idx0, idx1, idx2, idx3, ttv0, ttv1, ttv2, ttv3,
          rows0, rows1, ob0, ob1,
          pos_v, ttrow_v, gamma_v, beta_v,
          gsem0, gsem1, isem0, isem1, isem2, isem3, osem0, osem1):
    idx = (idx0, idx1, idx2, idx3)
    ttv = (ttv0, ttv1, ttv2, ttv3)
    rows = (rows0, rows1)
    obuf = (ob0, ob1)
    gsem = (gsem0, gsem1)
    isem = (isem0, isem1, isem2, isem3)
    osem = (osem0, osem1)

    wid = lax.axis_index("s") * NC + lax.axis_index("c")
    base_tile = wid * TOK_PER_TILE

    pltpu.sync_copy(pos_hbm, pos_v)
    pltpu.sync_copy(ttrow_hbm, ttrow_v)
    pltpu.sync_copy(gamma_hbm, gamma_v)
    pltpu.sync_copy(beta_hbm, beta_v)

    def ids_copies(c, ib):
        base = base_tile + c * T
        return (pltpu.make_async_copy(ids_hbm.at[pl.ds(base, T)],
                                      idx[ib], isem[ib]),
                pltpu.make_async_copy(tt_hbm.at[pl.ds(base, T)],
                                      ttv[ib].at[pl.ds(0, T)], isem[ib]))

    def gather(ib, rb):
        return pltpu.make_async_copy(word_hbm.at[idx[ib]], rows[rb], gsem[rb])

    def outcp(c, b):
        base = base_tile + c * T
        return pltpu.make_async_copy(obuf[b], out_hbm.at[pl.ds(base, T)],
                                     osem[b])

    def compute(rb, ib, c):
        rows_v, ttv_v, ob_v = rows[rb], ttv[ib], obuf[rb]
        s_off = lax.rem(c * T, S)
        gammas = [gamma_v[pl.ds(16 * k, 16)] for k in range(DV)]
        betas = [beta_v[pl.ds(16 * k, 16)] for k in range(DV)]

        @plsc.parallel_loop(0, T, step=TU)
        def _(i0):
            tgrp = ttv_v[pl.ds(i0, 16)]
            for u in range(TU):
                i = i0 + u
                t = tgrp[u]
                s_row = s_off + i
                xs = []
                for k in range(DV):
                    sl = pl.ds(16 * k, 16)
                    x = rows_v[i, sl] + pos_v[s_row, sl] + ttrow_v[t, sl]
                    xs.append(x)
                s1 = jnp.sum(_tree_sum(xs))
                s2 = jnp.sum(_tree_sum([x * x for x in xs]))
                mean = s1 * (1.0 / D)
                var = s2 * (1.0 / D) - mean * mean
                a = _rsqrt(jnp.full((16,), var + 1e-5, jnp.float32))
                sh = jnp.full((16,), -mean, jnp.float32) * a
                for k in range(DV):
                    y = (xs[k] * a + sh) * gammas[k] + betas[k]
                    ob_v[i, pl.ds(16 * k, 16)] = y

    def step(c, b4):
        rb = b4 % 2

        @pl.when(c + 1 < CHUNKS)
        def _():
            for cp in ids_copies(c + 1, (b4 + 1) % 4):
                cp.wait()
            gather((b4 + 1) % 4, 1 - rb).start()

        gather(b4, rb).wait()

        @pl.when(c >= 2)
        def _():
            outcp(c - 2, rb).wait()

        compute(rb, b4, c)

        @pl.when(c + 3 < CHUNKS)
        def _():
            for cp in ids_copies(c + 3, (b4 + 3) % 4):
                cp.start()

        outcp(c, rb).start()

    for k in (0, 1, 2):
        for cp in ids_copies(k, k):
            cp.start()
    for cp in ids_copies(0, 0):
        cp.wait()
    gather(0, 0).start()

    def loop_body(i, carry):
        for b4 in range(4):
            step(4 * i + b4, b4)
        return carry

    lax.fori_loop(0, CHUNKS // 4, loop_body, 0)
    outcp(CHUNKS - 2, 0).wait()
    outcp(CHUNKS - 1, 1).wait()


def kernel(input_ids, token_type_ids, word_table, pos_table, tt_table,
           ln_gamma, ln_beta):
    ids = input_ids.reshape(N).astype(jnp.int32)
    tts = token_type_ids.reshape(N).astype(jnp.int32)

    mesh = plsc.VectorSubcoreMesh(core_axis_name="c", subcore_axis_name="s",
                                  num_cores=NC, num_subcores=NS)
    fn = pl.kernel(
        _body,
        out_type=jax.ShapeDtypeStruct((N, D), jnp.float32),
        mesh=mesh,
        scratch_types=[
            pltpu.VMEM((T,), jnp.int32),
            pltpu.VMEM((T,), jnp.int32),
            pltpu.VMEM((T,), jnp.int32),
            pltpu.VMEM((T,), jnp.int32),
            pltpu.VMEM((T + 16,), jnp.int32),
            pltpu.VMEM((T + 16,), jnp.int32),
            pltpu.VMEM((T + 16,), jnp.int32),
            pltpu.VMEM((T + 16,), jnp.int32),
            pltpu.VMEM((T, D), jnp.float32),
            pltpu.VMEM((T, D), jnp.float32),
            pltpu.VMEM((T, D), jnp.float32),
            pltpu.VMEM((T, D), jnp.float32),
            pltpu.VMEM((S, D), jnp.float32),
            pltpu.VMEM((2, D), jnp.float32),
            pltpu.VMEM((D,), jnp.float32),
            pltpu.VMEM((D,), jnp.float32),
            pltpu.SemaphoreType.DMA,
            pltpu.SemaphoreType.DMA,
            pltpu.SemaphoreType.DMA,
            pltpu.SemaphoreType.DMA,
            pltpu.SemaphoreType.DMA,
            pltpu.SemaphoreType.DMA,
            pltpu.SemaphoreType.DMA,
            pltpu.SemaphoreType.DMA,
        ],
        compiler_params=pltpu.CompilerParams(needs_layout_passes=False),
    )
    out = fn(ids, tts, word_table, pos_table, tt_table, ln_gamma, ln_beta)
    return out.reshape(B, S, D)
